# baseline (speedup 1.0000x reference)
import jax
import jax.numpy as jnp
from jax import lax
from jax.experimental import pallas as pl
from jax.experimental.pallas import tpu as pltpu

B = 256
N_CLIN = 38
N_PIX = 36
FV = 128
SPB = 128
GRID = B // SPB


def _fused_kernel(clin_ref, img_ref, wg_ref, w39_ref, bias_ref, out_ref):
    clin3 = clin_ref[...]
    img3 = img_ref[...]
    wg = wg_ref[...]
    w39 = w39_ref[...].reshape(N_CLIN + 1, FV)

    s_clin = jnp.sum(clin3, axis=0)
    s_pix = jnp.sum(img3, axis=0)

    agg_c = clin3 + s_pix[None, :, :]
    agg_i = img3 + s_clin[None, :, :]

    dn = (((2,), (0,)), ((), ()))
    h_c = jnp.maximum(lax.dot_general(agg_c, wg, dn,
                                      preferred_element_type=jnp.float32), 0.0)
    h_i = jnp.maximum(lax.dot_general(agg_i, wg, dn,
                                      preferred_element_type=jnp.float32), 0.0)

    t = (h_c * w39[:N_CLIN, None, :]).sum(axis=0) \
        + (h_i * (w39[N_CLIN:, None, :] * (1.0 / N_PIX))).sum(axis=0)

    out = t.sum(axis=1) + bias_ref[0, 0]
    out_ref[...] = out[None, :]


def kernel(clinical_embeddings, image_embeddings, edge_index, W_g, W_out, b_out):
    del edge_index
    clin_nm = jnp.transpose(clinical_embeddings, (1, 0, 2))
    img_nm = jnp.transpose(image_embeddings, (1, 0, 2))
    w39 = W_out.reshape((N_CLIN + 1) * FV)
    bias = b_out.reshape(1, 1)
    fixed2 = lambda i: (0, 0)
    out = pl.pallas_call(
        _fused_kernel,
        grid=(GRID,),
        in_specs=[
            pl.BlockSpec((N_CLIN, SPB, FV), lambda i: (0, i, 0)),
            pl.BlockSpec((N_PIX, SPB, FV), lambda i: (0, i, 0)),
            pl.BlockSpec((FV, FV), fixed2),
            pl.BlockSpec(((N_CLIN + 1) * FV,), lambda i: (0,)),
            pl.BlockSpec((1, 1), fixed2),
        ],
        out_specs=pl.BlockSpec((1, SPB), lambda i: (0, i)),
        out_shape=jax.ShapeDtypeStruct((1, B), jnp.float32),
        compiler_params=pltpu.CompilerParams(
            dimension_semantics=("parallel",),
        ),
    )(clin_nm, img_nm, W_g, w39, bias)
    return jnp.transpose(out, (1, 0))
